# baseline (speedup 1.0000x reference)
import functools

import jax
import jax.numpy as jnp
from jax import lax
from jax.experimental import pallas as pl
from jax.experimental.pallas import tpu as pltpu
from jax.experimental.pallas import tpu_sc as plsc

N = 10000
E = 640000
D = 128
DU = D // 2
D_EDGE = 16

NC = 2
NS = 16
NW = NC * NS
CHUNK = 128
E_PAD = 655360
EPT = E_PAD // NW
NSTEP = EPT // CHUNK
N_PAD = 10112
ROWS_PT = N_PAD // NS
NVR = D // 16


_BE = 4000


def _ea_body(attr_ref, w_ref, b_ref, out_ref):
    ea = (
        jnp.dot(attr_ref[...], w_ref[...], preferred_element_type=jnp.float32)
        + b_ref[...]
    )
    lo = lax.bitcast_convert_type(
        ea[:, :DU].astype(jnp.bfloat16), jnp.uint16
    ).astype(jnp.uint32)
    hi = lax.bitcast_convert_type(
        ea[:, DU:].astype(jnp.bfloat16), jnp.uint16
    ).astype(jnp.uint32)
    out_ref[...] = lo | (hi << 16)


def _edge_encode(edge_attr, w_edge, b_edge):
    grid = (E // _BE,)
    return pl.pallas_call(
        _ea_body,
        grid=grid,
        in_specs=[
            pl.BlockSpec((_BE, D_EDGE), lambda i: (i, 0)),
            pl.BlockSpec((D_EDGE, D), lambda i: (0, 0)),
            pl.BlockSpec((1, D), lambda i: (0, 0)),
        ],
        out_specs=pl.BlockSpec((_BE, DU), lambda i: (i, 0)),
        out_shape=jax.ShapeDtypeStruct((E, DU), jnp.uint32),
    )(edge_attr, w_edge, b_edge.reshape(1, D))


_HI_MASK = jnp.uint32(0xFFFF0000)


def _sc_aggregate(idx3, ea, x):
    mesh = plsc.VectorSubcoreMesh(core_axis_name="c", subcore_axis_name="s")

    def body(idx_hbm, ea_hbm, x_hbm, out_hbm,
             idx_v0, idx_v1, ea_v0, ea_v1, m_v0, m_v1, acc_shared,
             s_idx0, s_idx1, s_ea0, s_ea1, s_g0, s_g1):
        cid = lax.axis_index("c")
        sid = lax.axis_index("s")
        wid = sid * NC + cid
        row0 = sid * ROWS_PT
        idx_v = (idx_v0, idx_v1)
        ea_v = (ea_v0, ea_v1)
        m_v = (m_v0, m_v1)
        s_idx = (s_idx0, s_idx1)
        s_ea = (s_ea0, s_ea1)
        s_g = (s_g0, s_g1)

        zero = jnp.zeros((16,), jnp.float32)

        def _zrow(r, _):
            for j in range(NVR):
                m_v0[r, pl.ds(j * 16, 16)] = zero
            return 0

        lax.fori_loop(0, CHUNK, _zrow, 0)
        for k in range(4):
            pltpu.sync_copy(m_v0, acc_shared.at[pl.ds(row0 + k * CHUNK, CHUNK)])
        pltpu.sync_copy(
            m_v0.at[pl.ds(0, 120)],
            acc_shared.at[pl.ds(row0 + 4 * CHUNK, 120)],
        )
        plsc.subcore_barrier()

        def _issue_linear(i, b):
            pltpu.async_copy(idx_hbm.at[wid, i], idx_v[b], s_idx[b])
            eab = pl.multiple_of(
                lax.min((wid * EPT + i * CHUNK) // 2, (E - CHUNK) // 2), 8
            )
            pltpu.async_copy(ea_hbm.at[pl.ds(eab, CHUNK // 2)], ea_v[b], s_ea[b])

        def _wait_idx(b):
            pltpu.make_async_copy(idx_hbm.at[0, 0], idx_v[b], s_idx[b]).wait()

        def _issue_gather(b):
            pltpu.async_copy(x_hbm.at[idx_v[b].at[0]], m_v[b], s_g[b])

        def _consume(i, b, prefetch_l, prefetch_g):
            if prefetch_g:
                _wait_idx(1 - b)
                _issue_gather(1 - b)
            pltpu.make_async_copy(
                ea_hbm.at[pl.ds(0, CHUNK // 2)], ea_v[b], s_ea[b]
            ).wait()
            pltpu.make_async_copy(x_hbm.at[idx_v[b].at[0]], m_v[b], s_g[b]).wait()

            def _edge(r, _):
                for half in range(2):
                    row = 2 * r + half
                    for t in range(4):
                        u = ea_v[b][r, pl.ds(half * DU + t * 16, 16)]
                        elo = plsc.bitcast(u << 16, jnp.float32)
                        ehi = plsc.bitcast(u & _HI_MASK, jnp.float32)
                        sl_lo = pl.ds(t * 16, 16)
                        sl_hi = pl.ds((4 + t) * 16, 16)
                        m_v[b][row, sl_lo] = jnp.maximum(m_v[b][row, sl_lo] + elo, 0.0)
                        m_v[b][row, sl_hi] = jnp.maximum(m_v[b][row, sl_hi] + ehi, 0.0)
                return 0

            lax.fori_loop(0, CHUNK // 2, _edge, 0)
            pltpu.sync_copy(m_v[b], acc_shared.at[idx_v[b].at[1]], add=True)
            if prefetch_l:
                _issue_linear(i + 2, b)

        _issue_linear(0, 0)
        _issue_linear(1, 1)
        _wait_idx(0)
        _issue_gather(0)

        def _steady(k, _):
            _consume(2 * k, 0, True, True)
            _consume(2 * k + 1, 1, True, True)
            return 0

        lax.fori_loop(0, NSTEP // 2 - 1, _steady, 0)
        _consume(NSTEP - 2, 0, False, True)
        _consume(NSTEP - 1, 1, False, False)

        plsc.subcore_barrier()

        for k in range(4):
            r = row0 + k * CHUNK
            buf = m_v[k % 2]
            pltpu.sync_copy(acc_shared.at[pl.ds(r, CHUNK)], buf)
            pltpu.sync_copy(buf, out_hbm.at[cid, pl.ds(r, CHUNK)])
        r = row0 + 4 * CHUNK
        pltpu.sync_copy(acc_shared.at[pl.ds(r, 120)], m_v0.at[pl.ds(0, 120)])
        pltpu.sync_copy(m_v0.at[pl.ds(0, 120)], out_hbm.at[cid, pl.ds(r, 120)])

    kern = pl.kernel(
        body,
        out_type=jax.ShapeDtypeStruct((NC, N_PAD, D), jnp.float32),
        mesh=mesh,
        compiler_params=pltpu.CompilerParams(needs_layout_passes=False),
        scratch_types=[
            pltpu.VMEM((2, CHUNK), jnp.int32),
            pltpu.VMEM((2, CHUNK), jnp.int32),
            pltpu.VMEM((CHUNK // 2, D), jnp.uint32),
            pltpu.VMEM((CHUNK // 2, D), jnp.uint32),
            pltpu.VMEM((CHUNK, D), jnp.float32),
            pltpu.VMEM((CHUNK, D), jnp.float32),
            pltpu.VMEM_SHARED((N_PAD, D), jnp.float32),
            pltpu.SemaphoreType.DMA,
            pltpu.SemaphoreType.DMA,
            pltpu.SemaphoreType.DMA,
            pltpu.SemaphoreType.DMA,
            pltpu.SemaphoreType.DMA,
            pltpu.SemaphoreType.DMA,
        ],
    )
    return kern(idx3, ea, x)


_BN = 2000
_BN_SCALE = 1.0 / (1.0 + 1e-5) ** 0.5


def _mlp_body(x_ref, agg_ref, w1_ref, b1_ref, g1_ref, t1_ref,
              w2_ref, b2_ref, g2_ref, t2_ref, g3_ref, t3_ref, eps_ref, out_ref):
    eps = eps_ref[0]
    h = x_ref[...] * (1.0 + eps) + agg_ref[0] + agg_ref[1]
    s1 = g1_ref[...] * _BN_SCALE
    h = jnp.maximum(
        jnp.dot(h, w1_ref[...], preferred_element_type=jnp.float32) * s1
        + (b1_ref[...] * s1 + t1_ref[...]),
        0.0,
    )
    s2 = g2_ref[...] * _BN_SCALE
    h = jnp.maximum(
        jnp.dot(h, w2_ref[...], preferred_element_type=jnp.float32) * s2
        + (b2_ref[...] * s2 + t2_ref[...]),
        0.0,
    )
    out_ref[...] = jnp.maximum(h * (g3_ref[...] * _BN_SCALE) + t3_ref[...], 0.0)


def _mlp(x, agg, w1, b1, g1, t1, w2, b2, g2, t2, g3, t3, eps_gin):
    grid = (N // _BN,)
    row = lambda a: a.reshape(1, D)
    full = pl.BlockSpec((1, D), lambda i: (0, 0))
    return pl.pallas_call(
        _mlp_body,
        grid=grid,
        in_specs=[
            pl.BlockSpec((_BN, D), lambda i: (i, 0)),
            pl.BlockSpec((NC, _BN, D), lambda i: (0, i, 0)),
            pl.BlockSpec((D, D), lambda i: (0, 0)),
            full, full, full,
            pl.BlockSpec((D, D), lambda i: (0, 0)),
            full, full, full, full, full,
            pl.BlockSpec(memory_space=pltpu.SMEM),
        ],
        out_specs=pl.BlockSpec((_BN, D), lambda i: (i, 0)),
        out_shape=jax.ShapeDtypeStruct((N, D), jnp.float32),
    )(x, agg, w1, row(b1), row(g1), row(t1),
      w2, row(b2), row(g2), row(t2), row(g3), row(t3),
      eps_gin.reshape(1))


def kernel(x, edge_index, edge_attr, W_edge, b_edge, W1, b1, g1, bt1,
           W2, b2, g2, bt2, g3, bt3, eps_gin):
    ei = edge_index.astype(jnp.int32)
    pad = jnp.zeros((2, E_PAD - E), jnp.int32).at[1, :].set(N)
    idx3 = (
        jnp.concatenate([ei, pad], axis=1)
        .reshape(2, NW, NSTEP, CHUNK)
        .transpose(1, 2, 0, 3)
    )
    ea = _edge_encode(edge_attr, W_edge, b_edge).reshape(E // 2, D)
    agg = _sc_aggregate(idx3, ea, x)
    return _mlp(x, agg, W1, b1, g1, bt1, W2, b2, g2, bt2, g3, bt3, eps_gin)


# speedup vs baseline: 1.2710x; 1.2710x over previous
import functools

import jax
import jax.numpy as jnp
from jax import lax
from jax.experimental import pallas as pl
from jax.experimental.pallas import tpu as pltpu
from jax.experimental.pallas import tpu_sc as plsc

N = 10000
E = 640000
D = 128
D_EDGE = 16

NC = 2
NS = 16
NW = NC * NS
EPT = E // NW
CHUNK = 80
NSTEP = EPT // CHUNK
N_PAD = 10240
ROWS_PT = N_PAD // NS
RCHUNK = 128
NVR = D // 16


_BE = 4000


def _ea_body(attr_ref, w_ref, b_ref, out_ref):
    out_ref[...] = (
        jnp.dot(attr_ref[...], w_ref[...], preferred_element_type=jnp.float32)
        + b_ref[...]
    )


def _edge_encode(edge_attr, w_edge, b_edge):
    grid = (E // _BE,)
    return pl.pallas_call(
        _ea_body,
        grid=grid,
        in_specs=[
            pl.BlockSpec((_BE, D_EDGE), lambda i: (i, 0)),
            pl.BlockSpec((D_EDGE, D), lambda i: (0, 0)),
            pl.BlockSpec((1, D), lambda i: (0, 0)),
        ],
        out_specs=pl.BlockSpec((_BE, D), lambda i: (i, 0)),
        out_shape=jax.ShapeDtypeStruct((E, D), jnp.float32),
    )(edge_attr, w_edge, b_edge.reshape(1, D))


def _sc_body(src_hbm, dst_hbm, ea_hbm, x_hbm, out_hbm,
             src_v, dst_v, ea_v, m_v, stage_v, acc_shared, sem):
    cid = lax.axis_index("c")
    sid = lax.axis_index("s")
    row0 = sid * ROWS_PT

    zero = jnp.zeros((16,), jnp.float32)

    def _zrow(r, _):
        for j in range(NVR):
            stage_v[r, pl.ds(j * 16, 16)] = zero
        return 0

    lax.fori_loop(0, RCHUNK, _zrow, 0)
    for k in range(ROWS_PT // RCHUNK):
        pltpu.sync_copy(stage_v, acc_shared.at[pl.ds(row0 + k * RCHUNK, RCHUNK)])
    plsc.subcore_barrier()

    tile_base = (sid * NC + cid) * EPT

    def _step(i, _):
        base = tile_base + i * CHUNK
        pltpu.sync_copy(src_hbm.at[pl.ds(base, CHUNK)], src_v)
        pltpu.sync_copy(dst_hbm.at[pl.ds(base, CHUNK)], dst_v)
        pltpu.sync_copy(ea_hbm.at[pl.ds(base, CHUNK)], ea_v)
        pltpu.async_copy(x_hbm.at[src_v], m_v, sem).wait()

        def _edge(e, _):
            for j in range(NVR):
                sl = pl.ds(j * 16, 16)
                m_v[e, sl] = jnp.maximum(m_v[e, sl] + ea_v[e, sl], 0.0)
            return 0

        lax.fori_loop(0, CHUNK, _edge, 0)
        pltpu.sync_copy(m_v, acc_shared.at[dst_v], add=True)
        return 0

    lax.fori_loop(0, NSTEP, _step, 0)
    plsc.subcore_barrier()

    for k in range(ROWS_PT // RCHUNK):
        r = row0 + k * RCHUNK
        pltpu.sync_copy(acc_shared.at[pl.ds(r, RCHUNK)], stage_v)
        pltpu.sync_copy(stage_v, out_hbm.at[cid, pl.ds(r, RCHUNK)])


def _sc_aggregate(src, dst, ea, x):
    mesh = plsc.VectorSubcoreMesh(core_axis_name="c", subcore_axis_name="s")
    kern = pl.kernel(
        _sc_body,
        out_type=jax.ShapeDtypeStruct((NC, N_PAD, D), jnp.float32),
        mesh=mesh,
        scratch_types=[
            pltpu.VMEM((CHUNK,), jnp.int32),
            pltpu.VMEM((CHUNK,), jnp.int32),
            pltpu.VMEM((CHUNK, D), jnp.float32),
            pltpu.VMEM((CHUNK, D), jnp.float32),
            pltpu.VMEM((RCHUNK, D), jnp.float32),
            pltpu.VMEM_SHARED((N_PAD, D), jnp.float32),
            pltpu.SemaphoreType.DMA,
        ],
    )
    return kern(src, dst, ea, x)


_BN = 2000
_BN_SCALE = 1.0 / (1.0 + 1e-5) ** 0.5


def _mlp_body(x_ref, agg_ref, w1_ref, b1_ref, g1_ref, t1_ref,
              w2_ref, b2_ref, g2_ref, t2_ref, g3_ref, t3_ref, eps_ref, out_ref):
    eps = eps_ref[0]
    h = x_ref[...] * (1.0 + eps) + agg_ref[0] + agg_ref[1]
    s1 = g1_ref[...] * _BN_SCALE
    h = jnp.maximum(
        jnp.dot(h, w1_ref[...], preferred_element_type=jnp.float32) * s1
        + (b1_ref[...] * s1 + t1_ref[...]),
        0.0,
    )
    s2 = g2_ref[...] * _BN_SCALE
    h = jnp.maximum(
        jnp.dot(h, w2_ref[...], preferred_element_type=jnp.float32) * s2
        + (b2_ref[...] * s2 + t2_ref[...]),
        0.0,
    )
    out_ref[...] = jnp.maximum(h * (g3_ref[...] * _BN_SCALE) + t3_ref[...], 0.0)


def _mlp(x, agg, w1, b1, g1, t1, w2, b2, g2, t2, g3, t3, eps_gin):
    grid = (N // _BN,)
    row = lambda a: a.reshape(1, D)
    full = pl.BlockSpec((1, D), lambda i: (0, 0))
    return pl.pallas_call(
        _mlp_body,
        grid=grid,
        in_specs=[
            pl.BlockSpec((_BN, D), lambda i: (i, 0)),
            pl.BlockSpec((NC, _BN, D), lambda i: (0, i, 0)),
            pl.BlockSpec((D, D), lambda i: (0, 0)),
            full, full, full,
            pl.BlockSpec((D, D), lambda i: (0, 0)),
            full, full, full, full, full,
            pl.BlockSpec(memory_space=pltpu.SMEM),
        ],
        out_specs=pl.BlockSpec((_BN, D), lambda i: (i, 0)),
        out_shape=jax.ShapeDtypeStruct((N, D), jnp.float32),
    )(x, agg, w1, row(b1), row(g1), row(t1),
      w2, row(b2), row(g2), row(t2), row(g3), row(t3),
      eps_gin.reshape(1))


def kernel(x, edge_index, edge_attr, W_edge, b_edge, W1, b1, g1, bt1,
           W2, b2, g2, bt2, g3, bt3, eps_gin):
    src = edge_index[0].astype(jnp.int32)
    dst = edge_index[1].astype(jnp.int32)
    ea = _edge_encode(edge_attr, W_edge, b_edge)
    agg = _sc_aggregate(src, dst, ea, x)
    return _mlp(x, agg, W1, b1, g1, bt1, W2, b2, g2, bt2, g3, bt3, eps_gin)


# speedup vs baseline: 1.3588x; 1.0691x over previous
import functools

import jax
import jax.numpy as jnp
from jax import lax
from jax.experimental import pallas as pl
from jax.experimental.pallas import tpu as pltpu
from jax.experimental.pallas import tpu_sc as plsc

N = 10000
E = 640000
D = 128
D_EDGE = 16

NC = 2
NS = 16
NW = NC * NS
E2 = E // 2
EPT = E2 // NW
CHUNK = 80
NSTEP = EPT // CHUNK
N_PAD = 10240
ROWS_PT = N_PAD // NS
RCHUNK = 128
NVR = D // 16


_BE = 4000


def _ea_body(attr_ref, w_ref, b_ref, out_ref):
    out_ref[...] = (
        jnp.dot(attr_ref[...], w_ref[...], preferred_element_type=jnp.float32)
        + b_ref[...]
    )


def _edge_encode(edge_attr, w_edge, b_edge):
    grid = (E2 // _BE,)
    return pl.pallas_call(
        _ea_body,
        grid=grid,
        in_specs=[
            pl.BlockSpec((_BE, D_EDGE), lambda i: (i, 0)),
            pl.BlockSpec((D_EDGE, D), lambda i: (0, 0)),
            pl.BlockSpec((1, D), lambda i: (0, 0)),
        ],
        out_specs=pl.BlockSpec((_BE, D), lambda i: (i, 0)),
        out_shape=jax.ShapeDtypeStruct((E2, D), jnp.float32),
    )(edge_attr, w_edge, b_edge.reshape(1, D))


def _sc_body(src_hbm, dst_hbm, ea_hbm, x_hbm, out_hbm,
             src_v, dst_v, ea_v, m_v, stage_v, acc_shared, sem):
    cid = lax.axis_index("c")
    sid = lax.axis_index("s")
    row0 = sid * ROWS_PT

    zero = jnp.zeros((16,), jnp.float32)

    def _zrow(r, _):
        for j in range(NVR):
            stage_v[r, pl.ds(j * 16, 16)] = zero
        return 0

    lax.fori_loop(0, RCHUNK, _zrow, 0)
    for k in range(ROWS_PT // RCHUNK):
        pltpu.sync_copy(stage_v, acc_shared.at[pl.ds(row0 + k * RCHUNK, RCHUNK)])
    plsc.subcore_barrier()

    tile_base = (sid * NC + cid) * EPT

    def _step(i, _):
        base = tile_base + i * CHUNK
        pltpu.sync_copy(src_hbm.at[pl.ds(base, CHUNK)], src_v)
        pltpu.sync_copy(dst_hbm.at[pl.ds(base, CHUNK)], dst_v)
        pltpu.sync_copy(ea_hbm.at[pl.ds(base, CHUNK)], ea_v)
        pltpu.async_copy(x_hbm.at[src_v], m_v, sem).wait()

        def _edge(e, _):
            for j in range(NVR):
                sl = pl.ds(j * 16, 16)
                m_v[e, sl] = jnp.maximum(m_v[e, sl] + ea_v[e, sl], 0.0)
            return 0

        lax.fori_loop(0, CHUNK, _edge, 0)
        pltpu.sync_copy(m_v, acc_shared.at[dst_v], add=True)
        return 0

    lax.fori_loop(0, NSTEP, _step, 0)
    plsc.subcore_barrier()

    for k in range(ROWS_PT // RCHUNK):
        r = row0 + k * RCHUNK
        pltpu.sync_copy(acc_shared.at[pl.ds(r, RCHUNK)], stage_v)
        pltpu.sync_copy(stage_v, out_hbm.at[cid, pl.ds(r, RCHUNK)])


def _sc_aggregate(src, dst, ea, x):
    mesh = plsc.VectorSubcoreMesh(core_axis_name="c", subcore_axis_name="s")
    kern = pl.kernel(
        _sc_body,
        out_type=jax.ShapeDtypeStruct((NC, N_PAD, D), jnp.float32),
        mesh=mesh,
        scratch_types=[
            pltpu.VMEM((CHUNK,), jnp.int32),
            pltpu.VMEM((CHUNK,), jnp.int32),
            pltpu.VMEM((CHUNK, D), jnp.float32),
            pltpu.VMEM((CHUNK, D), jnp.float32),
            pltpu.VMEM((RCHUNK, D), jnp.float32),
            pltpu.VMEM_SHARED((N_PAD, D), jnp.float32),
            pltpu.SemaphoreType.DMA,
        ],
    )
    return kern(src, dst, ea, x)


_BN = 2000
_BN_SCALE = 1.0 / (1.0 + 1e-5) ** 0.5


def _mlp_body(x_ref, agg_ref, agg2_ref, w1_ref, b1_ref, g1_ref, t1_ref,
              w2_ref, b2_ref, g2_ref, t2_ref, g3_ref, t3_ref, eps_ref, out_ref):
    eps = eps_ref[0]
    h = (x_ref[...] * (1.0 + eps) + (agg_ref[0] + agg_ref[1])
         + (agg2_ref[0] + agg2_ref[1]))
    s1 = g1_ref[...] * _BN_SCALE
    h = jnp.maximum(
        jnp.dot(h, w1_ref[...], preferred_element_type=jnp.float32) * s1
        + (b1_ref[...] * s1 + t1_ref[...]),
        0.0,
    )
    s2 = g2_ref[...] * _BN_SCALE
    h = jnp.maximum(
        jnp.dot(h, w2_ref[...], preferred_element_type=jnp.float32) * s2
        + (b2_ref[...] * s2 + t2_ref[...]),
        0.0,
    )
    out_ref[...] = jnp.maximum(h * (g3_ref[...] * _BN_SCALE) + t3_ref[...], 0.0)


def _mlp(x, agg, agg2, w1, b1, g1, t1, w2, b2, g2, t2, g3, t3, eps_gin):
    grid = (N // _BN,)
    row = lambda a: a.reshape(1, D)
    full = pl.BlockSpec((1, D), lambda i: (0, 0))
    return pl.pallas_call(
        _mlp_body,
        grid=grid,
        in_specs=[
            pl.BlockSpec((_BN, D), lambda i: (i, 0)),
            pl.BlockSpec((NC, _BN, D), lambda i: (0, i, 0)),
            pl.BlockSpec((NC, _BN, D), lambda i: (0, i, 0)),
            pl.BlockSpec((D, D), lambda i: (0, 0)),
            full, full, full,
            pl.BlockSpec((D, D), lambda i: (0, 0)),
            full, full, full, full, full,
            pl.BlockSpec(memory_space=pltpu.SMEM),
        ],
        out_specs=pl.BlockSpec((_BN, D), lambda i: (i, 0)),
        out_shape=jax.ShapeDtypeStruct((N, D), jnp.float32),
    )(x, agg, agg2, w1, row(b1), row(g1), row(t1),
      w2, row(b2), row(g2), row(t2), row(g3), row(t3),
      eps_gin.reshape(1))


def kernel(x, edge_index, edge_attr, W_edge, b_edge, W1, b1, g1, bt1,
           W2, b2, g2, bt2, g3, bt3, eps_gin):
    src = edge_index[0].astype(jnp.int32)
    dst = edge_index[1].astype(jnp.int32)
    ea1 = _edge_encode(edge_attr[:E2], W_edge, b_edge)
    ea2 = _edge_encode(edge_attr[E2:], W_edge, b_edge)
    agg1 = _sc_aggregate(src[:E2], dst[:E2], ea1, x)
    agg2 = _sc_aggregate(src[E2:], dst[E2:], ea2, x)
    return _mlp(x, agg1, agg2, W1, b1, g1, bt1, W2, b2, g2, bt2, g3, bt3,
                eps_gin)


# speedup vs baseline: 1.3707x; 1.0088x over previous
import functools

import jax
import jax.numpy as jnp
from jax import lax
from jax.experimental import pallas as pl
from jax.experimental.pallas import tpu as pltpu
from jax.experimental.pallas import tpu_sc as plsc

N = 10000
E = 640000
D = 128
D_EDGE = 16

NC = 2
NS = 16
NW = NC * NS
E_PARTS = (128000, 256000, 256000)
CHUNK = 80
N_PAD = 10240
ROWS_PT = N_PAD // NS
RCHUNK = 128
NVR = D // 16


_BE = 4000


def _ea_body(attr_ref, w_ref, b_ref, out_ref):
    out_ref[...] = (
        jnp.dot(attr_ref[...], w_ref[...], preferred_element_type=jnp.float32)
        + b_ref[...]
    )


def _edge_encode(edge_attr, w_edge, b_edge):
    e_part = edge_attr.shape[0]
    grid = (e_part // _BE,)
    return pl.pallas_call(
        _ea_body,
        grid=grid,
        in_specs=[
            pl.BlockSpec((_BE, D_EDGE), lambda i: (i, 0)),
            pl.BlockSpec((D_EDGE, D), lambda i: (0, 0)),
            pl.BlockSpec((1, D), lambda i: (0, 0)),
        ],
        out_specs=pl.BlockSpec((_BE, D), lambda i: (i, 0)),
        out_shape=jax.ShapeDtypeStruct((e_part, D), jnp.float32),
    )(edge_attr, w_edge, b_edge.reshape(1, D))


def _sc_body(ept, src_hbm, dst_hbm, ea_hbm, x_hbm, out_hbm,
             src_v, dst_v, ea_v, m_v, stage_v, acc_shared, sem):
    nstep = ept // CHUNK
    cid = lax.axis_index("c")
    sid = lax.axis_index("s")
    row0 = sid * ROWS_PT

    zero = jnp.zeros((16,), jnp.float32)

    def _zrow(r, _):
        for j in range(NVR):
            stage_v[r, pl.ds(j * 16, 16)] = zero
        return 0

    lax.fori_loop(0, RCHUNK, _zrow, 0)
    for k in range(ROWS_PT // RCHUNK):
        pltpu.sync_copy(stage_v, acc_shared.at[pl.ds(row0 + k * RCHUNK, RCHUNK)])
    plsc.subcore_barrier()

    tile_base = (sid * NC + cid) * ept

    def _step(i, _):
        base = tile_base + i * CHUNK
        pltpu.sync_copy(src_hbm.at[pl.ds(base, CHUNK)], src_v)
        pltpu.sync_copy(dst_hbm.at[pl.ds(base, CHUNK)], dst_v)
        pltpu.sync_copy(ea_hbm.at[pl.ds(base, CHUNK)], ea_v)
        pltpu.async_copy(x_hbm.at[src_v], m_v, sem).wait()

        def _edge(e, _):
            for j in range(NVR):
                sl = pl.ds(j * 16, 16)
                m_v[e, sl] = jnp.maximum(m_v[e, sl] + ea_v[e, sl], 0.0)
            return 0

        lax.fori_loop(0, CHUNK, _edge, 0)
        pltpu.sync_copy(m_v, acc_shared.at[dst_v], add=True)
        return 0

    lax.fori_loop(0, nstep, _step, 0)
    plsc.subcore_barrier()

    for k in range(ROWS_PT // RCHUNK):
        r = row0 + k * RCHUNK
        pltpu.sync_copy(acc_shared.at[pl.ds(r, RCHUNK)], stage_v)
        pltpu.sync_copy(stage_v, out_hbm.at[cid, pl.ds(r, RCHUNK)])


def _sc_aggregate(src, dst, ea, x):
    ept = src.shape[0] // NW
    mesh = plsc.VectorSubcoreMesh(core_axis_name="c", subcore_axis_name="s")
    kern = pl.kernel(
        functools.partial(_sc_body, ept),
        out_type=jax.ShapeDtypeStruct((NC, N_PAD, D), jnp.float32),
        mesh=mesh,
        scratch_types=[
            pltpu.VMEM((CHUNK,), jnp.int32),
            pltpu.VMEM((CHUNK,), jnp.int32),
            pltpu.VMEM((CHUNK, D), jnp.float32),
            pltpu.VMEM((CHUNK, D), jnp.float32),
            pltpu.VMEM((RCHUNK, D), jnp.float32),
            pltpu.VMEM_SHARED((N_PAD, D), jnp.float32),
            pltpu.SemaphoreType.DMA,
        ],
    )
    return kern(src, dst, ea, x)


_BN = 2000
_BN_SCALE = 1.0 / (1.0 + 1e-5) ** 0.5


def _mlp_body(x_ref, agg_ref, agg2_ref, agg3_ref, w1_ref, b1_ref, g1_ref,
              t1_ref, w2_ref, b2_ref, g2_ref, t2_ref, g3_ref, t3_ref,
              eps_ref, out_ref):
    eps = eps_ref[0]
    h = (x_ref[...] * (1.0 + eps) + (agg_ref[0] + agg_ref[1])
         + (agg2_ref[0] + agg2_ref[1]) + (agg3_ref[0] + agg3_ref[1]))
    s1 = g1_ref[...] * _BN_SCALE
    h = jnp.maximum(
        jnp.dot(h, w1_ref[...], preferred_element_type=jnp.float32) * s1
        + (b1_ref[...] * s1 + t1_ref[...]),
        0.0,
    )
    s2 = g2_ref[...] * _BN_SCALE
    h = jnp.maximum(
        jnp.dot(h, w2_ref[...], preferred_element_type=jnp.float32) * s2
        + (b2_ref[...] * s2 + t2_ref[...]),
        0.0,
    )
    out_ref[...] = jnp.maximum(h * (g3_ref[...] * _BN_SCALE) + t3_ref[...], 0.0)


def _mlp(x, aggs, w1, b1, g1, t1, w2, b2, g2, t2, g3, t3, eps_gin):
    grid = (N // _BN,)
    row = lambda a: a.reshape(1, D)
    full = pl.BlockSpec((1, D), lambda i: (0, 0))
    agg_spec = pl.BlockSpec((NC, _BN, D), lambda i: (0, i, 0))
    return pl.pallas_call(
        _mlp_body,
        grid=grid,
        in_specs=[
            pl.BlockSpec((_BN, D), lambda i: (i, 0)),
            agg_spec, agg_spec, agg_spec,
            pl.BlockSpec((D, D), lambda i: (0, 0)),
            full, full, full,
            pl.BlockSpec((D, D), lambda i: (0, 0)),
            full, full, full, full, full,
            pl.BlockSpec(memory_space=pltpu.SMEM),
        ],
        out_specs=pl.BlockSpec((_BN, D), lambda i: (i, 0)),
        out_shape=jax.ShapeDtypeStruct((N, D), jnp.float32),
    )(x, *aggs, w1, row(b1), row(g1), row(t1),
      w2, row(b2), row(g2), row(t2), row(g3), row(t3),
      eps_gin.reshape(1))


def kernel(x, edge_index, edge_attr, W_edge, b_edge, W1, b1, g1, bt1,
           W2, b2, g2, bt2, g3, bt3, eps_gin):
    src = edge_index[0].astype(jnp.int32)
    dst = edge_index[1].astype(jnp.int32)
    bounds = []
    lo = 0
    for sz in E_PARTS:
        bounds.append((lo, lo + sz))
        lo += sz
    eas = [_edge_encode(edge_attr[a:b], W_edge, b_edge) for a, b in bounds]
    aggs = [
        _sc_aggregate(src[a:b], dst[a:b], ea, x)
        for (a, b), ea in zip(bounds, eas)
    ]
    return _mlp(x, aggs, W1, b1, g1, bt1, W2, b2, g2, bt2, g3, bt3, eps_gin)


# speedup vs baseline: 1.3767x; 1.0044x over previous
import functools

import jax
import jax.numpy as jnp
from jax import lax
from jax.experimental import pallas as pl
from jax.experimental.pallas import tpu as pltpu
from jax.experimental.pallas import tpu_sc as plsc

N = 10000
E = 640000
D = 128
D_EDGE = 16

NC = 2
NS = 16
NW = NC * NS
E_PARTS = (64000, 192000, 192000, 192000)
CHUNK = 80
N_PAD = 10240
ROWS_PT = N_PAD // NS
RCHUNK = 128
NVR = D // 16


_BE = 4000


def _ea_body(attr_ref, w_ref, b_ref, out_ref):
    out_ref[...] = (
        jnp.dot(attr_ref[...], w_ref[...], preferred_element_type=jnp.float32)
        + b_ref[...]
    )


def _edge_encode(edge_attr, w_edge, b_edge):
    e_part = edge_attr.shape[0]
    grid = (e_part // _BE,)
    return pl.pallas_call(
        _ea_body,
        grid=grid,
        in_specs=[
            pl.BlockSpec((_BE, D_EDGE), lambda i: (i, 0)),
            pl.BlockSpec((D_EDGE, D), lambda i: (0, 0)),
            pl.BlockSpec((1, D), lambda i: (0, 0)),
        ],
        out_specs=pl.BlockSpec((_BE, D), lambda i: (i, 0)),
        out_shape=jax.ShapeDtypeStruct((e_part, D), jnp.float32),
    )(edge_attr, w_edge, b_edge.reshape(1, D))


def _sc_body(ept, src_hbm, dst_hbm, ea_hbm, x_hbm, out_hbm,
             src_v, dst_v, ea_v, m_v, stage_v, acc_shared, sem):
    nstep = ept // CHUNK
    cid = lax.axis_index("c")
    sid = lax.axis_index("s")
    row0 = sid * ROWS_PT

    zero = jnp.zeros((16,), jnp.float32)

    def _zrow(r, _):
        for j in range(NVR):
            stage_v[r, pl.ds(j * 16, 16)] = zero
        return 0

    lax.fori_loop(0, RCHUNK, _zrow, 0)
    for k in range(ROWS_PT // RCHUNK):
        pltpu.sync_copy(stage_v, acc_shared.at[pl.ds(row0 + k * RCHUNK, RCHUNK)])
    plsc.subcore_barrier()

    tile_base = (sid * NC + cid) * ept

    def _step(i, _):
        base = tile_base + i * CHUNK
        pltpu.sync_copy(src_hbm.at[pl.ds(base, CHUNK)], src_v)
        pltpu.sync_copy(dst_hbm.at[pl.ds(base, CHUNK)], dst_v)
        pltpu.sync_copy(ea_hbm.at[pl.ds(base, CHUNK)], ea_v)
        pltpu.async_copy(x_hbm.at[src_v], m_v, sem).wait()

        def _edge(e, _):
            for j in range(NVR):
                sl = pl.ds(j * 16, 16)
                m_v[e, sl] = jnp.maximum(m_v[e, sl] + ea_v[e, sl], 0.0)
            return 0

        lax.fori_loop(0, CHUNK, _edge, 0)
        pltpu.sync_copy(m_v, acc_shared.at[dst_v], add=True)
        return 0

    lax.fori_loop(0, nstep, _step, 0)
    plsc.subcore_barrier()

    for k in range(ROWS_PT // RCHUNK):
        r = row0 + k * RCHUNK
        pltpu.sync_copy(acc_shared.at[pl.ds(r, RCHUNK)], stage_v)
        pltpu.sync_copy(stage_v, out_hbm.at[cid, pl.ds(r, RCHUNK)])


def _sc_aggregate(src, dst, ea, x):
    ept = src.shape[0] // NW
    mesh = plsc.VectorSubcoreMesh(core_axis_name="c", subcore_axis_name="s")
    kern = pl.kernel(
        functools.partial(_sc_body, ept),
        out_type=jax.ShapeDtypeStruct((NC, N_PAD, D), jnp.float32),
        mesh=mesh,
        scratch_types=[
            pltpu.VMEM((CHUNK,), jnp.int32),
            pltpu.VMEM((CHUNK,), jnp.int32),
            pltpu.VMEM((CHUNK, D), jnp.float32),
            pltpu.VMEM((CHUNK, D), jnp.float32),
            pltpu.VMEM((RCHUNK, D), jnp.float32),
            pltpu.VMEM_SHARED((N_PAD, D), jnp.float32),
            pltpu.SemaphoreType.DMA,
        ],
    )
    return kern(src, dst, ea, x)


_BN = 2000
_BN_SCALE = 1.0 / (1.0 + 1e-5) ** 0.5


def _mlp_body(x_ref, agg_ref, agg2_ref, agg3_ref, agg4_ref, w1_ref, b1_ref,
              g1_ref, t1_ref, w2_ref, b2_ref, g2_ref, t2_ref, g3_ref, t3_ref,
              eps_ref, out_ref):
    eps = eps_ref[0]
    h = (x_ref[...] * (1.0 + eps) + (agg_ref[0] + agg_ref[1])
         + (agg2_ref[0] + agg2_ref[1]) + (agg3_ref[0] + agg3_ref[1])
         + (agg4_ref[0] + agg4_ref[1]))
    s1 = g1_ref[...] * _BN_SCALE
    h = jnp.maximum(
        jnp.dot(h, w1_ref[...], preferred_element_type=jnp.float32) * s1
        + (b1_ref[...] * s1 + t1_ref[...]),
        0.0,
    )
    s2 = g2_ref[...] * _BN_SCALE
    h = jnp.maximum(
        jnp.dot(h, w2_ref[...], preferred_element_type=jnp.float32) * s2
        + (b2_ref[...] * s2 + t2_ref[...]),
        0.0,
    )
    out_ref[...] = jnp.maximum(h * (g3_ref[...] * _BN_SCALE) + t3_ref[...], 0.0)


def _mlp(x, aggs, w1, b1, g1, t1, w2, b2, g2, t2, g3, t3, eps_gin):
    grid = (N // _BN,)
    row = lambda a: a.reshape(1, D)
    full = pl.BlockSpec((1, D), lambda i: (0, 0))
    agg_spec = pl.BlockSpec((NC, _BN, D), lambda i: (0, i, 0))
    return pl.pallas_call(
        _mlp_body,
        grid=grid,
        in_specs=[
            pl.BlockSpec((_BN, D), lambda i: (i, 0)),
            agg_spec, agg_spec, agg_spec, agg_spec,
            pl.BlockSpec((D, D), lambda i: (0, 0)),
            full, full, full,
            pl.BlockSpec((D, D), lambda i: (0, 0)),
            full, full, full, full, full,
            pl.BlockSpec(memory_space=pltpu.SMEM),
        ],
        out_specs=pl.BlockSpec((_BN, D), lambda i: (i, 0)),
        out_shape=jax.ShapeDtypeStruct((N, D), jnp.float32),
    )(x, *aggs, w1, row(b1), row(g1), row(t1),
      w2, row(b2), row(g2), row(t2), row(g3), row(t3),
      eps_gin.reshape(1))


def kernel(x, edge_index, edge_attr, W_edge, b_edge, W1, b1, g1, bt1,
           W2, b2, g2, bt2, g3, bt3, eps_gin):
    src = edge_index[0].astype(jnp.int32)
    dst = edge_index[1].astype(jnp.int32)
    bounds = []
    lo = 0
    for sz in E_PARTS:
        bounds.append((lo, lo + sz))
        lo += sz
    eas = [_edge_encode(edge_attr[a:b], W_edge, b_edge) for a, b in bounds]
    aggs = [
        _sc_aggregate(src[a:b], dst[a:b], ea, x)
        for (a, b), ea in zip(bounds, eas)
    ]
    return _mlp(x, aggs, W1, b1, g1, bt1, W2, b2, g2, bt2, g3, bt3, eps_gin)
